# baseline (speedup 1.0000x reference)
import functools

import jax
import jax.numpy as jnp
from jax import lax
from jax.experimental import pallas as pl
from jax.experimental.pallas import tpu as pltpu
from jax.experimental.pallas import tpu_sc as plsc

_IDS = [0, 195, 13, 182, 90, 110, 175, 6, 84, 45, 51, 129, 135, 69, 186, 3,
        9, 42, 48, 87, 93, 126, 132, 152, 192, 25, 81, 155, 159, 41, 53, 157,
        163, 184, 15, 18, 21, 30, 33, 36, 38, 57, 60, 63, 66, 72, 75, 78, 97,
        99, 102, 105, 108, 114, 117, 120, 123, 125, 142, 144, 147, 150, 165,
        167, 180, 188, 190, 1, 2, 4, 5, 7, 8, 10, 11, 12, 14, 16, 17, 19, 20,
        22, 23, 24, 26, 27, 28, 29, 31, 32, 34, 35, 37, 39, 40, 43, 44, 46,
        47, 49, 50, 52, 54, 55, 56, 58, 59, 61, 62, 64, 65, 67, 68, 70, 71,
        73, 74, 76, 77, 79, 80, 82, 83, 85, 86, 88, 89, 91, 92, 94, 95, 96,
        98, 100, 101, 103, 104, 106, 107, 109, 111, 112, 113, 115, 116, 118,
        119, 121, 122, 124, 127, 128, 130, 131, 133, 134, 136, 137, 138, 139,
        140, 141, 143, 145, 146, 148, 149, 151, 153, 154, 156, 158, 160, 161,
        162, 164, 166, 168, 169, 170, 171, 172, 173, 174, 176, 177, 178, 179,
        181, 183, 185, 187, 189, 191, 193, 194]


def _contiguous_runs(ids):
    runs = []
    j = 0
    while j < len(ids):
        a = ids[j]
        n = 1
        while j + n < len(ids) and ids[j + n] == a + n:
            n += 1
        runs.append((j, a, n))
        j += n
    return runs


def _assign(runs, num_workers):
    bins = [[] for _ in range(num_workers)]
    loads = [0] * num_workers
    for run in sorted(runs, key=lambda r: -r[2]):
        w = loads.index(min(loads))
        bins[w].append(run)
        loads[w] += run[2]
    return bins


_NB = 4
_CH = 48
_TAIL = 16


def _sc_body(in_hbm, ids48_hbm, ids16_hbm, oidx_hbm, out_hbm,
             idx48, idx16, oidx, vbuf, vtail, gsem, wsem):
    ncores = 2
    wid = lax.axis_index("s") * ncores + lax.axis_index("c")
    pltpu.sync_copy(ids48_hbm, idx48)
    pltpu.sync_copy(ids16_hbm, idx16)
    pltpu.sync_copy(oidx_hbm, oidx)
    for bi in range(_NB):
        b = wid * _NB + bi
        for c in range(4):
            pltpu.async_copy(
                in_hbm.at[b].at[idx48.at[c]], vbuf, gsem).wait()
            pltpu.async_copy(
                vbuf, out_hbm.at[b, pl.ds(c * _CH, _CH), :], wsem).wait()
        pltpu.async_copy(
            in_hbm.at[b].at[idx16], vtail, gsem).wait()
        pltpu.async_copy(
            vtail, out_hbm.at[b].at[oidx], wsem).wait()


def kernel(inputs):
    b, t, d = inputs.shape
    ids48 = jnp.asarray([_IDS[c * _CH:(c + 1) * _CH] for c in range(4)],
                        dtype=jnp.int32)
    ids16 = jnp.asarray(_IDS[t - _TAIL:], dtype=jnp.int32)
    oidx = jnp.asarray(list(range(t - _TAIL, t)), dtype=jnp.int32)
    mesh = plsc.VectorSubcoreMesh(core_axis_name="c", subcore_axis_name="s")
    run = functools.partial(
        pl.kernel,
        out_type=jax.ShapeDtypeStruct((b, t, d), inputs.dtype),
        mesh=mesh,
        scratch_types=[
            pltpu.VMEM((4, _CH), jnp.int32),
            pltpu.VMEM((_TAIL,), jnp.int32),
            pltpu.VMEM((_TAIL,), jnp.int32),
            pltpu.VMEM((_CH, 768), jnp.float32),
            pltpu.VMEM((_TAIL, 768), jnp.float32),
            pltpu.SemaphoreType.DMA,
            pltpu.SemaphoreType.DMA,
        ],
    )(_sc_body)
    return run(inputs, ids48, ids16, oidx)


# speedup vs baseline: 1.0568x; 1.0568x over previous
import functools

import jax
import jax.numpy as jnp
from jax import lax
from jax.experimental import pallas as pl
from jax.experimental.pallas import tpu as pltpu
from jax.experimental.pallas import tpu_sc as plsc

_IDS = [0, 195, 13, 182, 90, 110, 175, 6, 84, 45, 51, 129, 135, 69, 186, 3,
        9, 42, 48, 87, 93, 126, 132, 152, 192, 25, 81, 155, 159, 41, 53, 157,
        163, 184, 15, 18, 21, 30, 33, 36, 38, 57, 60, 63, 66, 72, 75, 78, 97,
        99, 102, 105, 108, 114, 117, 120, 123, 125, 142, 144, 147, 150, 165,
        167, 180, 188, 190, 1, 2, 4, 5, 7, 8, 10, 11, 12, 14, 16, 17, 19, 20,
        22, 23, 24, 26, 27, 28, 29, 31, 32, 34, 35, 37, 39, 40, 43, 44, 46,
        47, 49, 50, 52, 54, 55, 56, 58, 59, 61, 62, 64, 65, 67, 68, 70, 71,
        73, 74, 76, 77, 79, 80, 82, 83, 85, 86, 88, 89, 91, 92, 94, 95, 96,
        98, 100, 101, 103, 104, 106, 107, 109, 111, 112, 113, 115, 116, 118,
        119, 121, 122, 124, 127, 128, 130, 131, 133, 134, 136, 137, 138, 139,
        140, 141, 143, 145, 146, 148, 149, 151, 153, 154, 156, 158, 160, 161,
        162, 164, 166, 168, 169, 170, 171, 172, 173, 174, 176, 177, 178, 179,
        181, 183, 185, 187, 189, 191, 193, 194]


def _contiguous_runs(ids):
    runs = []
    j = 0
    while j < len(ids):
        a = ids[j]
        n = 1
        while j + n < len(ids) and ids[j + n] == a + n:
            n += 1
        runs.append((j, a, n))
        j += n
    return runs


def _assign(runs, num_workers):
    bins = [[] for _ in range(num_workers)]
    loads = [0] * num_workers
    for run in sorted(runs, key=lambda r: -r[2]):
        w = loads.index(min(loads))
        bins[w].append(run)
        loads[w] += run[2]
    return bins


_NB = 4
_CH = 48
_TAIL = 16


def _sc_body(in_hbm, ids48_hbm, ids16_hbm, oidx_hbm, out_hbm,
             idx48, idx16, oidx, vbufs, vtails, gsems, wsems):
    ncores = 2
    wid = lax.axis_index("s") * ncores + lax.axis_index("c")
    pltpu.sync_copy(ids48_hbm, idx48)
    pltpu.sync_copy(ids16_hbm, idx16)
    pltpu.sync_copy(oidx_hbm, oidx)

    tasks = [(bi, c) for bi in range(_NB) for c in range(4)]

    def g_copy(t, k):
        bi, c = tasks[t]
        return pltpu.make_async_copy(
            in_hbm.at[wid * _NB + bi].at[idx48.at[c]], vbufs.at[k],
            gsems.at[k])

    def w_copy(t, k):
        bi, c = tasks[t]
        return pltpu.make_async_copy(
            vbufs.at[k],
            out_hbm.at[wid * _NB + bi, pl.ds(c * _CH, _CH), :], wsems.at[k])

    n = len(tasks)
    g_copy(0, 0).start()
    for t in range(n):
        k = t % 2
        g_copy(t, k).wait()
        w_copy(t, k).start()
        if t + 1 < n:
            k2 = (t + 1) % 2
            if t >= 1:
                w_copy(t - 1, k2).wait()
            g_copy(t + 1, k2).start()
    w_copy(n - 2, n % 2).wait()
    w_copy(n - 1, (n - 1) % 2).wait()

    def gt_copy(bi, k):
        return pltpu.make_async_copy(
            in_hbm.at[wid * _NB + bi].at[idx16], vtails.at[k], gsems.at[k])

    def wt_copy(bi, k):
        return pltpu.make_async_copy(
            vtails.at[k], out_hbm.at[wid * _NB + bi].at[oidx], wsems.at[k])

    gt_copy(0, 0).start()
    for bi in range(_NB):
        k = bi % 2
        gt_copy(bi, k).wait()
        wt_copy(bi, k).start()
        if bi + 1 < _NB:
            k2 = (bi + 1) % 2
            if bi >= 1:
                wt_copy(bi - 1, k2).wait()
            gt_copy(bi + 1, k2).start()
    wt_copy(_NB - 2, _NB % 2).wait()
    wt_copy(_NB - 1, (_NB - 1) % 2).wait()


def kernel(inputs):
    b, t, d = inputs.shape
    ids48 = jnp.asarray([_IDS[c * _CH:(c + 1) * _CH] for c in range(4)],
                        dtype=jnp.int32)
    ids16 = jnp.asarray(_IDS[t - _TAIL:], dtype=jnp.int32)
    oidx = jnp.asarray(list(range(t - _TAIL, t)), dtype=jnp.int32)
    mesh = plsc.VectorSubcoreMesh(core_axis_name="c", subcore_axis_name="s")
    run = functools.partial(
        pl.kernel,
        out_type=jax.ShapeDtypeStruct((b, t, d), inputs.dtype),
        mesh=mesh,
        scratch_types=[
            pltpu.VMEM((4, _CH), jnp.int32),
            pltpu.VMEM((_TAIL,), jnp.int32),
            pltpu.VMEM((_TAIL,), jnp.int32),
            pltpu.VMEM((2, _CH, 768), jnp.float32),
            pltpu.VMEM((2, _TAIL, 768), jnp.float32),
            pltpu.SemaphoreType.DMA((2,)),
            pltpu.SemaphoreType.DMA((2,)),
        ],
    )(_sc_body)
    return run(inputs, ids48, ids16, oidx)
